# Initial kernel scaffold; baseline (speedup 1.0000x reference)
import jax
import jax.numpy as jnp
from jax.experimental import pallas as pl


def kernel(x, edge_index, W0, b0, Wconvs, Wout, bout):
    raise NotImplementedError("write your pallas kernel here")


# speedup vs baseline: 6.4836x; 6.4836x over previous
import functools
import math

import jax
import jax.numpy as jnp
from jax import lax
from jax.experimental import pallas as pl
from jax.experimental.pallas import tpu as pltpu
from jax.experimental.pallas import tpu_sc as plsc

N_NODES = 50000
N_EDGES = 800000
D_IN = 128
HID = 64
N_LAYERS = 8
ALPHA = 0.1
THETA = 0.5

F = HID // 2
TILES = 16
EPT = N_EDGES // TILES
DB = 125
DPC = 4
CH = DB * DPC
NCHUNK = EPT // CH
RPT_A = 3128
RPT_LAST = N_NODES - 15 * RPT_A

_mesh = plsc.VectorSubcoreMesh(core_axis_name="c", subcore_axis_name="s")


def _segsum_body(hlo, hhi, srcr, dstr, outlo, outhi, idx_s, idx_d, rows, acc, sem):
    c = lax.axis_index("c")
    s = lax.axis_index("s")
    base = pl.multiple_of(s * RPT_A, 8)

    zero16 = jnp.zeros((16,), jnp.float32)

    def _zero(i, _):
        rows[i, pl.ds(0, 16)] = zero16
        rows[i, pl.ds(16, 16)] = zero16
        return _

    lax.fori_loop(0, CH, _zero, None)
    nfull = RPT_LAST // CH
    for q in range(nfull):
        pltpu.sync_copy(rows, acc.at[pl.ds(base + q * CH, CH)])

    @pl.when(s < TILES - 1)
    def _():
        pltpu.sync_copy(rows.at[pl.ds(0, RPT_A - nfull * CH)], acc.at[pl.ds(base + nfull * CH, RPT_A - nfull * CH)])

    @pl.when(s == TILES - 1)
    def _():
        pltpu.sync_copy(rows.at[pl.ds(0, RPT_LAST - nfull * CH)], acc.at[pl.ds(base + nfull * CH, RPT_LAST - nfull * CH)])

    plsc.subcore_barrier()

    def _chunk(k, _):
        r0 = (s * NCHUNK + k) * DPC
        pltpu.sync_copy(srcr.at[pl.ds(r0, DPC)], idx_s)
        pltpu.sync_copy(dstr.at[pl.ds(r0, DPC)], idx_d)

        @pl.when(c == 0)
        def _():
            cps = [
                pltpu.async_copy(hlo.at[idx_s.at[j]], rows.at[pl.ds(j * DB, DB)], sem)
                for j in range(DPC)
            ]
            for cp in cps:
                cp.wait()

        @pl.when(c == 1)
        def _():
            cps = [
                pltpu.async_copy(hhi.at[idx_s.at[j]], rows.at[pl.ds(j * DB, DB)], sem)
                for j in range(DPC)
            ]
            for cp in cps:
                cp.wait()

        for j in range(DPC):
            pltpu.sync_copy(rows.at[pl.ds(j * DB, DB)], acc.at[idx_d.at[j]], add=True)
        return _

    lax.fori_loop(0, NCHUNK, _chunk, None)
    plsc.subcore_barrier()

    @pl.when(jnp.logical_and(c == 0, s < TILES - 1))
    def _():
        pltpu.sync_copy(acc.at[pl.ds(base, RPT_A)], outlo.at[pl.ds(base, RPT_A)])

    @pl.when(jnp.logical_and(c == 0, s == TILES - 1))
    def _():
        pltpu.sync_copy(acc.at[pl.ds(base, RPT_LAST)], outlo.at[pl.ds(base, RPT_LAST)])

    @pl.when(jnp.logical_and(c == 1, s < TILES - 1))
    def _():
        pltpu.sync_copy(acc.at[pl.ds(base, RPT_A)], outhi.at[pl.ds(base, RPT_A)])

    @pl.when(jnp.logical_and(c == 1, s == TILES - 1))
    def _():
        pltpu.sync_copy(acc.at[pl.ds(base, RPT_LAST)], outhi.at[pl.ds(base, RPT_LAST)])


_segsum = pl.kernel(
    _segsum_body,
    out_type=[
        jax.ShapeDtypeStruct((N_NODES, F), jnp.float32),
        jax.ShapeDtypeStruct((N_NODES, F), jnp.float32),
    ],
    mesh=_mesh,
    compiler_params=pltpu.CompilerParams(use_tc_tiling_on_sc=False),
    scratch_types=[
        pltpu.VMEM((DPC, DB), jnp.int32),
        pltpu.VMEM((DPC, DB), jnp.int32),
        pltpu.VMEM((CH, F), jnp.float32),
        pltpu.VMEM_SHARED((N_NODES, F), jnp.float32),
        pltpu.SemaphoreType.DMA,
    ],
)


_BR = 2000
_NB = N_NODES // _BR


def _proj_body(x_ref, w_ref, b_ref, ol_ref, oh_ref):
    h = jnp.dot(x_ref[...], w_ref[...], preferred_element_type=jnp.float32)
    h = jnp.maximum(h + b_ref[...], 0.0)
    ol_ref[...] = h[:, :F]
    oh_ref[...] = h[:, F:]


_proj = pl.pallas_call(
    _proj_body,
    grid=(_NB,),
    in_specs=[
        pl.BlockSpec((_BR, D_IN), lambda i: (i, 0)),
        pl.BlockSpec((D_IN, HID), lambda i: (0, 0)),
        pl.BlockSpec((1, HID), lambda i: (0, 0)),
    ],
    out_specs=[
        pl.BlockSpec((_BR, F), lambda i: (i, 0)),
        pl.BlockSpec((_BR, F), lambda i: (i, 0)),
    ],
    out_shape=[
        jax.ShapeDtypeStruct((N_NODES, F), jnp.float32),
        jax.ShapeDtypeStruct((N_NODES, F), jnp.float32),
    ],
)


def _dense_body(al_ref, ah_ref, hl_ref, hh_ref, w_ref, ol_ref, oh_ref):
    ml = (1.0 - ALPHA) * al_ref[...] + ALPHA * hl_ref[...]
    mh = (1.0 - ALPHA) * ah_ref[...] + ALPHA * hh_ref[...]
    r = jnp.dot(ml, w_ref[0], preferred_element_type=jnp.float32)
    r = r + jnp.dot(mh, w_ref[1], preferred_element_type=jnp.float32)
    r = jnp.maximum(r, 0.0)
    ol_ref[...] = r[:, :F]
    oh_ref[...] = r[:, F:]


_dense = pl.pallas_call(
    _dense_body,
    grid=(_NB,),
    in_specs=[
        pl.BlockSpec((_BR, F), lambda i: (i, 0)),
        pl.BlockSpec((_BR, F), lambda i: (i, 0)),
        pl.BlockSpec((_BR, F), lambda i: (i, 0)),
        pl.BlockSpec((_BR, F), lambda i: (i, 0)),
        pl.BlockSpec((2, F, HID), lambda i: (0, 0, 0)),
    ],
    out_specs=[
        pl.BlockSpec((_BR, F), lambda i: (i, 0)),
        pl.BlockSpec((_BR, F), lambda i: (i, 0)),
    ],
    out_shape=[
        jax.ShapeDtypeStruct((N_NODES, F), jnp.float32),
        jax.ShapeDtypeStruct((N_NODES, F), jnp.float32),
    ],
)


def _outp_body(hl_ref, hh_ref, w_ref, b_ref, o_ref):
    r = jnp.dot(hl_ref[...], w_ref[0], preferred_element_type=jnp.float32)
    r = r + jnp.dot(hh_ref[...], w_ref[1], preferred_element_type=jnp.float32)
    o_ref[...] = r + b_ref[...]


_outp = pl.pallas_call(
    _outp_body,
    grid=(_NB,),
    in_specs=[
        pl.BlockSpec((_BR, F), lambda i: (i, 0)),
        pl.BlockSpec((_BR, F), lambda i: (i, 0)),
        pl.BlockSpec((2, F, HID), lambda i: (0, 0, 0)),
        pl.BlockSpec((1, HID), lambda i: (0, 0)),
    ],
    out_specs=pl.BlockSpec((_BR, HID), lambda i: (i, 0)),
    out_shape=jax.ShapeDtypeStruct((N_NODES, HID), jnp.float32),
)


def kernel(x, edge_index, W0, b0, Wconvs, Wout, bout):
    ei = edge_index.astype(jnp.int32)
    srcr = ei[0].reshape(N_EDGES // DB, DB)
    dstr = ei[1].reshape(N_EDGES // DB, DB)

    betas = jnp.asarray(
        [math.log(THETA / (l + 1) + 1.0) for l in range(N_LAYERS)], jnp.float32
    )
    eye = jnp.eye(HID, dtype=jnp.float32)
    wp = (1.0 - betas)[:, None, None] * eye + betas[:, None, None] * Wconvs
    wp2 = jnp.stack([wp[:, :F, :], wp[:, F:, :]], axis=1)
    wout2 = jnp.stack([Wout[:F], Wout[F:]])

    hl, hh = _proj(x, W0, b0.reshape(1, HID))
    h0l, h0h = hl, hh
    for l in range(N_LAYERS):
        al, ah = _segsum(hl, hh, srcr, dstr)
        hl, hh = _dense(al, ah, h0l, h0h, wp2[l])
    return _outp(hl, hh, wout2, bout.reshape(1, HID))


# speedup vs baseline: 8.4615x; 1.3051x over previous
import functools
import math

import jax
import jax.numpy as jnp
from jax import lax
from jax.experimental import pallas as pl
from jax.experimental.pallas import tpu as pltpu
from jax.experimental.pallas import tpu_sc as plsc

N_NODES = 50000
N_EDGES = 800000
D_IN = 128
HID = 64
N_LAYERS = 8
ALPHA = 0.1
THETA = 0.5

F = HID // 2
TILES = 16
EPT = N_EDGES // TILES
DB = 125
DPC = 2
CH = DB * DPC
CPB = 20
IBR = 4 * DPC
NIB = CPB // 4
NBODY = EPT // (CH * CPB)
RPT_A = 3128
RPT_LAST = N_NODES - 15 * RPT_A

_mesh = plsc.VectorSubcoreMesh(core_axis_name="c", subcore_axis_name="s")


def _segsum_body(hlo, hhi, srcr, dstr, outlo, outhi,
                 idx_s0, idx_s1, idx_d0, idx_d1, rows0, rows1, acc,
                 gsem0, gsem1, ssem0, ssem1, isem0, isem1):
    c = lax.axis_index("c")
    s = lax.axis_index("s")
    base = pl.multiple_of(s * RPT_A, 8)
    isrc = (idx_s0, idx_s1)
    idst = (idx_d0, idx_d1)
    rbuf = (rows0, rows1)
    gsem = (gsem0, gsem1)
    ssem = (ssem0, ssem1)
    isem = (isem0, isem1)

    zero16 = jnp.zeros((16,), jnp.float32)

    def _zero(i, _):
        rows0[i, pl.ds(0, 16)] = zero16
        rows0[i, pl.ds(16, 16)] = zero16
        return _

    lax.fori_loop(0, CH, _zero, None)
    nfull = RPT_LAST // CH
    for q in range(nfull):
        pltpu.sync_copy(rows0, acc.at[pl.ds(base + q * CH, CH)])

    @pl.when(s < TILES - 1)
    def _():
        pltpu.sync_copy(rows0.at[pl.ds(0, RPT_A - nfull * CH)], acc.at[pl.ds(base + nfull * CH, RPT_A - nfull * CH)])

    @pl.when(s == TILES - 1)
    def _():
        pltpu.sync_copy(rows0.at[pl.ds(0, RPT_LAST - nfull * CH)], acc.at[pl.ds(base + nfull * CH, RPT_LAST - nfull * CH)])

    plsc.subcore_barrier()

    def _body(t, _):
        def idx_fire(m):
            ib = m & 1
            r0 = pl.multiple_of((s * (NIB * NBODY) + t * NIB + m) * IBR, 8)
            return [
                pltpu.async_copy(srcr.at[pl.ds(r0, IBR)], isrc[ib], isem[ib]),
                pltpu.async_copy(dstr.at[pl.ds(r0, IBR)], idst[ib], isem[ib]),
            ]

        ih = {0: idx_fire(0), 1: idx_fire(1)}

        def g_fire(cc, rb):
            m, k = divmod(cc, 4)
            ib = m & 1
            out = []

            @pl.when(c == 0)
            def _():
                for j in range(DPC):
                    out.append(pltpu.async_copy(
                        hlo.at[isrc[ib].at[k * DPC + j]],
                        rbuf[rb].at[pl.ds(j * DB, DB)], gsem[rb]))

            @pl.when(c == 1)
            def _():
                for j in range(DPC):
                    out.append(pltpu.async_copy(
                        hhi.at[isrc[ib].at[k * DPC + j]],
                        rbuf[rb].at[pl.ds(j * DB, DB)], gsem[rb]))

            return out

        def g_wait(rb):
            for j in range(DPC):
                pltpu.make_async_copy(
                    hlo.at[isrc[0].at[0]],
                    rbuf[rb].at[pl.ds(j * DB, DB)], gsem[rb]).wait()

        def s_fire(cc, rb):
            m, k = divmod(cc, 4)
            ib = m & 1
            return [pltpu.async_copy(
                rbuf[rb].at[pl.ds(j * DB, DB)],
                acc.at[idst[ib].at[k * DPC + j]], ssem[rb], add=True)
                for j in range(DPC)]

        def s_wait(rb):
            for j in range(DPC):
                pltpu.make_async_copy(
                    rbuf[rb].at[pl.ds(j * DB, DB)],
                    acc.at[idst[0].at[0]], ssem[rb]).wait()

        for cp in ih[0]:
            cp.wait()
        g_fire(0, 0)
        for cc in range(CPB):
            rb = cc & 1
            g_wait(rb)
            if cc + 1 < CPB:
                if (cc + 1) % 4 == 0:
                    for cp in ih[(cc + 1) // 4]:
                        cp.wait()
                if cc >= 1:
                    s_wait(1 - rb)
                if cc % 4 == 0 and 0 < cc and cc // 4 + 1 < NIB:
                    ih[cc // 4 + 1] = idx_fire(cc // 4 + 1)
                g_fire(cc + 1, 1 - rb)
            s_fire(cc, rb)
        s_wait(0)
        s_wait(1)
        return _

    lax.fori_loop(0, NBODY, _body, None)
    plsc.subcore_barrier()

    @pl.when(jnp.logical_and(c == 0, s < TILES - 1))
    def _():
        pltpu.sync_copy(acc.at[pl.ds(base, RPT_A)], outlo.at[pl.ds(base, RPT_A)])

    @pl.when(jnp.logical_and(c == 0, s == TILES - 1))
    def _():
        pltpu.sync_copy(acc.at[pl.ds(base, RPT_LAST)], outlo.at[pl.ds(base, RPT_LAST)])

    @pl.when(jnp.logical_and(c == 1, s < TILES - 1))
    def _():
        pltpu.sync_copy(acc.at[pl.ds(base, RPT_A)], outhi.at[pl.ds(base, RPT_A)])

    @pl.when(jnp.logical_and(c == 1, s == TILES - 1))
    def _():
        pltpu.sync_copy(acc.at[pl.ds(base, RPT_LAST)], outhi.at[pl.ds(base, RPT_LAST)])


_segsum = pl.kernel(
    _segsum_body,
    out_type=[
        jax.ShapeDtypeStruct((N_NODES, F), jnp.float32),
        jax.ShapeDtypeStruct((N_NODES, F), jnp.float32),
    ],
    mesh=_mesh,
    compiler_params=pltpu.CompilerParams(use_tc_tiling_on_sc=False),
    scratch_types=[
        pltpu.VMEM((IBR, DB), jnp.int32),
        pltpu.VMEM((IBR, DB), jnp.int32),
        pltpu.VMEM((IBR, DB), jnp.int32),
        pltpu.VMEM((IBR, DB), jnp.int32),
        pltpu.VMEM((CH, F), jnp.float32),
        pltpu.VMEM((CH, F), jnp.float32),
        pltpu.VMEM_SHARED((N_NODES, F), jnp.float32),
        pltpu.SemaphoreType.DMA,
        pltpu.SemaphoreType.DMA,
        pltpu.SemaphoreType.DMA,
        pltpu.SemaphoreType.DMA,
        pltpu.SemaphoreType.DMA,
        pltpu.SemaphoreType.DMA,
    ],
)


_BR = 2000
_NB = N_NODES // _BR


def _proj_body(x_ref, w_ref, b_ref, ol_ref, oh_ref):
    h = jnp.dot(x_ref[...], w_ref[...], preferred_element_type=jnp.float32)
    h = jnp.maximum(h + b_ref[...], 0.0)
    ol_ref[...] = h[:, :F]
    oh_ref[...] = h[:, F:]


_proj = pl.pallas_call(
    _proj_body,
    grid=(_NB,),
    in_specs=[
        pl.BlockSpec((_BR, D_IN), lambda i: (i, 0)),
        pl.BlockSpec((D_IN, HID), lambda i: (0, 0)),
        pl.BlockSpec((1, HID), lambda i: (0, 0)),
    ],
    out_specs=[
        pl.BlockSpec((_BR, F), lambda i: (i, 0)),
        pl.BlockSpec((_BR, F), lambda i: (i, 0)),
    ],
    out_shape=[
        jax.ShapeDtypeStruct((N_NODES, F), jnp.float32),
        jax.ShapeDtypeStruct((N_NODES, F), jnp.float32),
    ],
)


def _dense_body(al_ref, ah_ref, hl_ref, hh_ref, w_ref, ol_ref, oh_ref):
    ml = (1.0 - ALPHA) * al_ref[...] + ALPHA * hl_ref[...]
    mh = (1.0 - ALPHA) * ah_ref[...] + ALPHA * hh_ref[...]
    r = jnp.dot(ml, w_ref[0], preferred_element_type=jnp.float32)
    r = r + jnp.dot(mh, w_ref[1], preferred_element_type=jnp.float32)
    r = jnp.maximum(r, 0.0)
    ol_ref[...] = r[:, :F]
    oh_ref[...] = r[:, F:]


_dense = pl.pallas_call(
    _dense_body,
    grid=(_NB,),
    in_specs=[
        pl.BlockSpec((_BR, F), lambda i: (i, 0)),
        pl.BlockSpec((_BR, F), lambda i: (i, 0)),
        pl.BlockSpec((_BR, F), lambda i: (i, 0)),
        pl.BlockSpec((_BR, F), lambda i: (i, 0)),
        pl.BlockSpec((2, F, HID), lambda i: (0, 0, 0)),
    ],
    out_specs=[
        pl.BlockSpec((_BR, F), lambda i: (i, 0)),
        pl.BlockSpec((_BR, F), lambda i: (i, 0)),
    ],
    out_shape=[
        jax.ShapeDtypeStruct((N_NODES, F), jnp.float32),
        jax.ShapeDtypeStruct((N_NODES, F), jnp.float32),
    ],
)


def _outp_body(hl_ref, hh_ref, w_ref, b_ref, o_ref):
    r = jnp.dot(hl_ref[...], w_ref[0], preferred_element_type=jnp.float32)
    r = r + jnp.dot(hh_ref[...], w_ref[1], preferred_element_type=jnp.float32)
    o_ref[...] = r + b_ref[...]


_outp = pl.pallas_call(
    _outp_body,
    grid=(_NB,),
    in_specs=[
        pl.BlockSpec((_BR, F), lambda i: (i, 0)),
        pl.BlockSpec((_BR, F), lambda i: (i, 0)),
        pl.BlockSpec((2, F, HID), lambda i: (0, 0, 0)),
        pl.BlockSpec((1, HID), lambda i: (0, 0)),
    ],
    out_specs=pl.BlockSpec((_BR, HID), lambda i: (i, 0)),
    out_shape=jax.ShapeDtypeStruct((N_NODES, HID), jnp.float32),
)


def kernel(x, edge_index, W0, b0, Wconvs, Wout, bout):
    ei = edge_index.astype(jnp.int32)
    srcr = ei[0].reshape(N_EDGES // DB, DB)
    dstr = ei[1].reshape(N_EDGES // DB, DB)

    betas = jnp.asarray(
        [math.log(THETA / (l + 1) + 1.0) for l in range(N_LAYERS)], jnp.float32
    )
    eye = jnp.eye(HID, dtype=jnp.float32)
    wp = (1.0 - betas)[:, None, None] * eye + betas[:, None, None] * Wconvs
    wp2 = jnp.stack([wp[:, :F, :], wp[:, F:, :]], axis=1)
    wout2 = jnp.stack([Wout[:F], Wout[F:]])

    hl, hh = _proj(x, W0, b0.reshape(1, HID))
    h0l, h0h = hl, hh
    for l in range(N_LAYERS):
        al, ah = _segsum(hl, hh, srcr, dstr)
        hl, hh = _dense(al, ah, h0l, h0h, wp2[l])
    return _outp(hl, hh, wout2, bout.reshape(1, HID))


# speedup vs baseline: 17.1206x; 2.0233x over previous
import functools
import math

import jax
import jax.numpy as jnp
from jax import lax
from jax.experimental import pallas as pl
from jax.experimental.pallas import tpu as pltpu
from jax.experimental.pallas import tpu_sc as plsc

N_NODES = 50000
N_EDGES = 800000
D_IN = 128
HID = 64
N_LAYERS = 8
ALPHA = 0.1
THETA = 0.5

F = HID // 2
TILES = 16
EPT = N_EDGES // TILES
DB = 125
DPC = 2
CH = DB * DPC
CPB = 20
IBR = 4 * DPC
NIB = CPB // 4
NBODY = EPT // (CH * CPB)
RPT_A = 3128
RPT_LAST = N_NODES - 15 * RPT_A

_mesh = plsc.VectorSubcoreMesh(core_axis_name="c", subcore_axis_name="s")


def _segsum_body(hlo, hhi, srcr, dstr, outlo, outhi,
                 idx_s0, idx_s1, idx_d0, idx_d1, rows0, rows1, rows2, acc,
                 gsem0, gsem1, gsem2, ssem0, ssem1, ssem2, isem0, isem1):
    c = lax.axis_index("c")
    s = lax.axis_index("s")
    base = pl.multiple_of(s * RPT_A, 8)
    isrc = (idx_s0, idx_s1)
    idst = (idx_d0, idx_d1)
    rbuf = (rows0, rows1, rows2)
    gsem = (gsem0, gsem1, gsem2)
    ssem = (ssem0, ssem1, ssem2)
    isem = (isem0, isem1)

    zero16 = jnp.zeros((16,), jnp.float32)

    def _zero(i, _):
        rows0[i, pl.ds(0, 16)] = zero16
        rows0[i, pl.ds(16, 16)] = zero16
        return _

    lax.fori_loop(0, CH, _zero, None)
    nfull = RPT_LAST // CH
    for q in range(nfull):
        pltpu.sync_copy(rows0, acc.at[pl.ds(base + q * CH, CH)])

    @pl.when(s < TILES - 1)
    def _():
        pltpu.sync_copy(rows0.at[pl.ds(0, RPT_A - nfull * CH)], acc.at[pl.ds(base + nfull * CH, RPT_A - nfull * CH)])

    @pl.when(s == TILES - 1)
    def _():
        pltpu.sync_copy(rows0.at[pl.ds(0, RPT_LAST - nfull * CH)], acc.at[pl.ds(base + nfull * CH, RPT_LAST - nfull * CH)])

    plsc.subcore_barrier()

    def _body(t, _):
        def idx_fire(m):
            ib = m & 1
            r0 = pl.multiple_of((s * (NIB * NBODY) + t * NIB + m) * IBR, 8)
            return [
                pltpu.async_copy(srcr.at[pl.ds(r0, IBR)], isrc[ib], isem[ib]),
                pltpu.async_copy(dstr.at[pl.ds(r0, IBR)], idst[ib], isem[ib]),
            ]

        ih = {0: idx_fire(0), 1: idx_fire(1)}

        def g_fire(cc, rb):
            m, k = divmod(cc, 4)
            ib = m & 1
            out = []

            @pl.when(c == 0)
            def _():
                for j in range(DPC):
                    out.append(pltpu.async_copy(
                        hlo.at[isrc[ib].at[k * DPC + j]],
                        rbuf[rb].at[pl.ds(j * DB, DB)], gsem[rb]))

            @pl.when(c == 1)
            def _():
                for j in range(DPC):
                    out.append(pltpu.async_copy(
                        hhi.at[isrc[ib].at[k * DPC + j]],
                        rbuf[rb].at[pl.ds(j * DB, DB)], gsem[rb]))

            return out

        def g_wait(rb):
            for j in range(DPC):
                pltpu.make_async_copy(
                    hlo.at[isrc[0].at[0]],
                    rbuf[rb].at[pl.ds(j * DB, DB)], gsem[rb]).wait()

        def s_fire(cc, rb):
            m, k = divmod(cc, 4)
            ib = m & 1
            return [pltpu.async_copy(
                rbuf[rb].at[pl.ds(j * DB, DB)],
                acc.at[idst[ib].at[k * DPC + j]], ssem[rb], add=True)
                for j in range(DPC)]

        def s_wait(rb):
            for j in range(DPC):
                pltpu.make_async_copy(
                    rbuf[rb].at[pl.ds(j * DB, DB)],
                    acc.at[idst[0].at[0]], ssem[rb]).wait()

        for cp in ih[0]:
            cp.wait()
        g_fire(0, 0)
        g_fire(1, 1)
        for cc in range(CPB):
            rb = cc % 3
            if cc >= 1:
                s_wait((cc - 1) % 3)
            if cc + 2 < CPB:
                if (cc + 2) % 4 == 0:
                    for cp in ih[(cc + 2) // 4]:
                        cp.wait()
                if cc % 4 == 0 and 0 < cc and cc // 4 + 1 < NIB:
                    ih[cc // 4 + 1] = idx_fire(cc // 4 + 1)
                g_fire(cc + 2, (cc + 2) % 3)
            g_wait(rb)
            s_fire(cc, rb)
        s_wait((CPB - 1) % 3)
        return _

    lax.fori_loop(0, NBODY, _body, None)
    plsc.subcore_barrier()

    @pl.when(jnp.logical_and(c == 0, s < TILES - 1))
    def _():
        pltpu.sync_copy(acc.at[pl.ds(base, RPT_A)], outlo.at[pl.ds(base, RPT_A)])

    @pl.when(jnp.logical_and(c == 0, s == TILES - 1))
    def _():
        pltpu.sync_copy(acc.at[pl.ds(base, RPT_LAST)], outlo.at[pl.ds(base, RPT_LAST)])

    @pl.when(jnp.logical_and(c == 1, s < TILES - 1))
    def _():
        pltpu.sync_copy(acc.at[pl.ds(base, RPT_A)], outhi.at[pl.ds(base, RPT_A)])

    @pl.when(jnp.logical_and(c == 1, s == TILES - 1))
    def _():
        pltpu.sync_copy(acc.at[pl.ds(base, RPT_LAST)], outhi.at[pl.ds(base, RPT_LAST)])


_segsum = pl.kernel(
    _segsum_body,
    out_type=[
        jax.ShapeDtypeStruct((N_NODES, F), jnp.float32),
        jax.ShapeDtypeStruct((N_NODES, F), jnp.float32),
    ],
    mesh=_mesh,
    compiler_params=pltpu.CompilerParams(use_tc_tiling_on_sc=False),
    scratch_types=[
        pltpu.VMEM((IBR, DB), jnp.int32),
        pltpu.VMEM((IBR, DB), jnp.int32),
        pltpu.VMEM((IBR, DB), jnp.int32),
        pltpu.VMEM((IBR, DB), jnp.int32),
        pltpu.VMEM((CH, F), jnp.float32),
        pltpu.VMEM((CH, F), jnp.float32),
        pltpu.VMEM((CH, F), jnp.float32),
        pltpu.VMEM_SHARED((N_NODES, F), jnp.float32),
        pltpu.SemaphoreType.DMA,
        pltpu.SemaphoreType.DMA,
        pltpu.SemaphoreType.DMA,
        pltpu.SemaphoreType.DMA,
        pltpu.SemaphoreType.DMA,
        pltpu.SemaphoreType.DMA,
        pltpu.SemaphoreType.DMA,
        pltpu.SemaphoreType.DMA,
    ],
)


NQ = N_NODES // 4
_BQ = 1000
_NBQ = -(-NQ // _BQ)
_BR = 4 * _BQ
_NB = _NBQ


def _proj_body(x_ref, w_ref, b_ref, ol_ref, oh_ref):
    xb = x_ref[...]
    rl = jnp.dot(xb, w_ref[0], preferred_element_type=jnp.float32)
    rh = jnp.dot(xb, w_ref[1], preferred_element_type=jnp.float32)
    ol_ref[...] = jnp.maximum(rl + b_ref[0:1, :], 0.0)
    oh_ref[...] = jnp.maximum(rh + b_ref[1:2, :], 0.0)


_proj = pl.pallas_call(
    _proj_body,
    grid=(_NBQ,),
    in_specs=[
        pl.BlockSpec((_BQ, 4 * D_IN), lambda i: (i, 0)),
        pl.BlockSpec((2, 4 * D_IN, 4 * F), lambda i: (0, 0, 0)),
        pl.BlockSpec((2, 4 * F), lambda i: (0, 0)),
    ],
    out_specs=[
        pl.BlockSpec((_BQ, 4 * F), lambda i: (i, 0)),
        pl.BlockSpec((_BQ, 4 * F), lambda i: (i, 0)),
    ],
    out_shape=[
        jax.ShapeDtypeStruct((NQ, 4 * F), jnp.float32),
        jax.ShapeDtypeStruct((NQ, 4 * F), jnp.float32),
    ],
)


def _dense_body(al_ref, ah_ref, hl_ref, hh_ref, w_ref, ol_ref, oh_ref):
    ml = (1.0 - ALPHA) * al_ref[...] + ALPHA * hl_ref[...]
    mh = (1.0 - ALPHA) * ah_ref[...] + ALPHA * hh_ref[...]
    rl = jnp.dot(ml, w_ref[0], preferred_element_type=jnp.float32)
    rl = rl + jnp.dot(mh, w_ref[1], preferred_element_type=jnp.float32)
    rh = jnp.dot(ml, w_ref[2], preferred_element_type=jnp.float32)
    rh = rh + jnp.dot(mh, w_ref[3], preferred_element_type=jnp.float32)
    ol_ref[...] = jnp.maximum(rl, 0.0)
    oh_ref[...] = jnp.maximum(rh, 0.0)


_dense = pl.pallas_call(
    _dense_body,
    grid=(_NBQ,),
    in_specs=[
        pl.BlockSpec((_BQ, 4 * F), lambda i: (i, 0)),
        pl.BlockSpec((_BQ, 4 * F), lambda i: (i, 0)),
        pl.BlockSpec((_BQ, 4 * F), lambda i: (i, 0)),
        pl.BlockSpec((_BQ, 4 * F), lambda i: (i, 0)),
        pl.BlockSpec((4, 4 * F, 4 * F), lambda i: (0, 0, 0)),
    ],
    out_specs=[
        pl.BlockSpec((_BQ, 4 * F), lambda i: (i, 0)),
        pl.BlockSpec((_BQ, 4 * F), lambda i: (i, 0)),
    ],
    out_shape=[
        jax.ShapeDtypeStruct((NQ, 4 * F), jnp.float32),
        jax.ShapeDtypeStruct((NQ, 4 * F), jnp.float32),
    ],
)


def _outp_body(hl_ref, hh_ref, w_ref, b_ref, o_ref):
    r = jnp.dot(hl_ref[...], w_ref[0], preferred_element_type=jnp.float32)
    r = r + jnp.dot(hh_ref[...], w_ref[1], preferred_element_type=jnp.float32)
    o_ref[...] = r + b_ref[...]


_outp = pl.pallas_call(
    _outp_body,
    grid=(_NBQ,),
    in_specs=[
        pl.BlockSpec((_BQ, 4 * F), lambda i: (i, 0)),
        pl.BlockSpec((_BQ, 4 * F), lambda i: (i, 0)),
        pl.BlockSpec((2, 4 * F, 4 * HID), lambda i: (0, 0, 0)),
        pl.BlockSpec((1, 4 * HID), lambda i: (0, 0)),
    ],
    out_specs=pl.BlockSpec((_BQ, 4 * HID), lambda i: (i, 0)),
    out_shape=jax.ShapeDtypeStruct((NQ, 4 * HID), jnp.float32),
)


def kernel(x, edge_index, W0, b0, Wconvs, Wout, bout):
    ei = edge_index.astype(jnp.int32)
    srcr = ei[0].reshape(N_EDGES // DB, DB)
    dstr = ei[1].reshape(N_EDGES // DB, DB)

    betas = jnp.asarray(
        [math.log(THETA / (l + 1) + 1.0) for l in range(N_LAYERS)], jnp.float32
    )
    eye = jnp.eye(HID, dtype=jnp.float32)
    wp = (1.0 - betas)[:, None, None] * eye + betas[:, None, None] * Wconvs
    eye4 = jnp.eye(4, dtype=jnp.float32)
    wd = jnp.stack(
        [
            jnp.stack(
                [
                    jnp.kron(eye4, wp[l, :F, :F]),
                    jnp.kron(eye4, wp[l, F:, :F]),
                    jnp.kron(eye4, wp[l, :F, F:]),
                    jnp.kron(eye4, wp[l, F:, F:]),
                ]
            )
            for l in range(N_LAYERS)
        ]
    )
    wpj = jnp.stack(
        [jnp.kron(eye4, W0[:, :F]), jnp.kron(eye4, W0[:, F:])]
    )
    bpj = jnp.stack([jnp.tile(b0[:F], 4), jnp.tile(b0[F:], 4)])
    wo4 = jnp.stack([jnp.kron(eye4, Wout[:F]), jnp.kron(eye4, Wout[F:])])
    bo4 = jnp.tile(bout, 4).reshape(1, 4 * HID)

    x4 = x.reshape(NQ, 4 * D_IN)
    hl4, hh4 = _proj(x4, wpj, bpj)
    h0l4, h0h4 = hl4, hh4
    for l in range(N_LAYERS):
        al, ah = _segsum(
            hl4.reshape(N_NODES, F), hh4.reshape(N_NODES, F), srcr, dstr
        )
        hl4, hh4 = _dense(
            al.reshape(NQ, 4 * F), ah.reshape(NQ, 4 * F), h0l4, h0h4, wd[l]
        )
    return _outp(hl4, hh4, wo4, bo4).reshape(N_NODES, HID)


# speedup vs baseline: 18.5939x; 1.0861x over previous
import functools
import math

import jax
import jax.numpy as jnp
from jax import lax
from jax.experimental import pallas as pl
from jax.experimental.pallas import tpu as pltpu
from jax.experimental.pallas import tpu_sc as plsc

N_NODES = 50000
N_EDGES = 800000
D_IN = 128
HID = 64
N_LAYERS = 8
ALPHA = 0.1
THETA = 0.5

F = HID // 2
TILES = 16
EPT = N_EDGES // TILES
DB = 125
DPC = 2
CH = DB * DPC
CPB = 40
IBR = 4 * DPC
NIB = CPB // 4
NBODY = EPT // (CH * CPB)
RPT_A = 3128
RPT_LAST = N_NODES - 15 * RPT_A

_mesh = plsc.VectorSubcoreMesh(core_axis_name="c", subcore_axis_name="s")


def _segsum_body(hlo, hhi, srcr, dstr, outlo, outhi,
                 idx_s0, idx_s1, idx_d0, idx_d1, rows0, rows1, rows2, acc,
                 gsem0, gsem1, gsem2, ssem0, ssem1, ssem2, isem0, isem1):
    c = lax.axis_index("c")
    s = lax.axis_index("s")
    base = pl.multiple_of(s * RPT_A, 8)
    isrc = (idx_s0, idx_s1)
    idst = (idx_d0, idx_d1)
    rbuf = (rows0, rows1, rows2)
    gsem = (gsem0, gsem1, gsem2)
    ssem = (ssem0, ssem1, ssem2)
    isem = (isem0, isem1)

    zero16 = jnp.zeros((16,), jnp.float32)

    def _zero(i, _):
        rows0[i, pl.ds(0, 16)] = zero16
        rows0[i, pl.ds(16, 16)] = zero16
        return _

    lax.fori_loop(0, CH, _zero, None)
    nfull = RPT_LAST // CH
    for q in range(nfull):
        pltpu.sync_copy(rows0, acc.at[pl.ds(base + q * CH, CH)])

    @pl.when(s < TILES - 1)
    def _():
        pltpu.sync_copy(rows0.at[pl.ds(0, RPT_A - nfull * CH)], acc.at[pl.ds(base + nfull * CH, RPT_A - nfull * CH)])

    @pl.when(s == TILES - 1)
    def _():
        pltpu.sync_copy(rows0.at[pl.ds(0, RPT_LAST - nfull * CH)], acc.at[pl.ds(base + nfull * CH, RPT_LAST - nfull * CH)])

    plsc.subcore_barrier()

    def _body(t, _):
        def idx_fire(m):
            ib = m & 1
            r0 = pl.multiple_of((s * (NIB * NBODY) + t * NIB + m) * IBR, 8)
            return [
                pltpu.async_copy(srcr.at[pl.ds(r0, IBR)], isrc[ib], isem[ib]),
                pltpu.async_copy(dstr.at[pl.ds(r0, IBR)], idst[ib], isem[ib]),
            ]

        ih = {0: idx_fire(0), 1: idx_fire(1)}

        def g_fire(cc, rb):
            m, k = divmod(cc, 4)
            ib = m & 1
            out = []

            @pl.when(c == 0)
            def _():
                for j in range(DPC):
                    out.append(pltpu.async_copy(
                        hlo.at[isrc[ib].at[k * DPC + j]],
                        rbuf[rb].at[pl.ds(j * DB, DB)], gsem[rb]))

            @pl.when(c == 1)
            def _():
                for j in range(DPC):
                    out.append(pltpu.async_copy(
                        hhi.at[isrc[ib].at[k * DPC + j]],
                        rbuf[rb].at[pl.ds(j * DB, DB)], gsem[rb]))

            return out

        def g_wait(rb):
            for j in range(DPC):
                pltpu.make_async_copy(
                    hlo.at[isrc[0].at[0]],
                    rbuf[rb].at[pl.ds(j * DB, DB)], gsem[rb]).wait()

        def s_fire(cc, rb):
            m, k = divmod(cc, 4)
            ib = m & 1
            return [pltpu.async_copy(
                rbuf[rb].at[pl.ds(j * DB, DB)],
                acc.at[idst[ib].at[k * DPC + j]], ssem[rb], add=True)
                for j in range(DPC)]

        def s_wait(rb):
            for j in range(DPC):
                pltpu.make_async_copy(
                    rbuf[rb].at[pl.ds(j * DB, DB)],
                    acc.at[idst[0].at[0]], ssem[rb]).wait()

        for cp in ih[0]:
            cp.wait()
        g_fire(0, 0)
        g_fire(1, 1)
        for cc in range(CPB):
            rb = cc % 3
            if cc >= 1:
                s_wait((cc - 1) % 3)
            if cc + 2 < CPB:
                if (cc + 2) % 4 == 0:
                    for cp in ih[(cc + 2) // 4]:
                        cp.wait()
                if cc % 4 == 0 and 0 < cc and cc // 4 + 1 < NIB:
                    ih[cc // 4 + 1] = idx_fire(cc // 4 + 1)
                g_fire(cc + 2, (cc + 2) % 3)
            g_wait(rb)
            s_fire(cc, rb)
        s_wait((CPB - 1) % 3)
        return _

    lax.fori_loop(0, NBODY, _body, None)
    plsc.subcore_barrier()

    @pl.when(jnp.logical_and(c == 0, s < TILES - 1))
    def _():
        pltpu.sync_copy(acc.at[pl.ds(base, RPT_A)], outlo.at[pl.ds(base, RPT_A)])

    @pl.when(jnp.logical_and(c == 0, s == TILES - 1))
    def _():
        pltpu.sync_copy(acc.at[pl.ds(base, RPT_LAST)], outlo.at[pl.ds(base, RPT_LAST)])

    @pl.when(jnp.logical_and(c == 1, s < TILES - 1))
    def _():
        pltpu.sync_copy(acc.at[pl.ds(base, RPT_A)], outhi.at[pl.ds(base, RPT_A)])

    @pl.when(jnp.logical_and(c == 1, s == TILES - 1))
    def _():
        pltpu.sync_copy(acc.at[pl.ds(base, RPT_LAST)], outhi.at[pl.ds(base, RPT_LAST)])


_segsum = pl.kernel(
    _segsum_body,
    out_type=[
        jax.ShapeDtypeStruct((N_NODES, F), jnp.float32),
        jax.ShapeDtypeStruct((N_NODES, F), jnp.float32),
    ],
    mesh=_mesh,
    compiler_params=pltpu.CompilerParams(use_tc_tiling_on_sc=False),
    scratch_types=[
        pltpu.VMEM((IBR, DB), jnp.int32),
        pltpu.VMEM((IBR, DB), jnp.int32),
        pltpu.VMEM((IBR, DB), jnp.int32),
        pltpu.VMEM((IBR, DB), jnp.int32),
        pltpu.VMEM((CH, F), jnp.float32),
        pltpu.VMEM((CH, F), jnp.float32),
        pltpu.VMEM((CH, F), jnp.float32),
        pltpu.VMEM_SHARED((N_NODES, F), jnp.float32),
        pltpu.SemaphoreType.DMA,
        pltpu.SemaphoreType.DMA,
        pltpu.SemaphoreType.DMA,
        pltpu.SemaphoreType.DMA,
        pltpu.SemaphoreType.DMA,
        pltpu.SemaphoreType.DMA,
        pltpu.SemaphoreType.DMA,
        pltpu.SemaphoreType.DMA,
    ],
)


NQ = N_NODES // 4
_BQ = 2000
_NBQ = -(-NQ // _BQ)
_BR = 4 * _BQ
_NB = _NBQ


def _proj_body(x_ref, w_ref, b_ref, ol_ref, oh_ref):
    xb = x_ref[...]
    rl = jnp.dot(xb, w_ref[0], preferred_element_type=jnp.float32)
    rh = jnp.dot(xb, w_ref[1], preferred_element_type=jnp.float32)
    ol_ref[...] = jnp.maximum(rl + b_ref[0:1, :], 0.0)
    oh_ref[...] = jnp.maximum(rh + b_ref[1:2, :], 0.0)


_proj = pl.pallas_call(
    _proj_body,
    grid=(_NBQ,),
    in_specs=[
        pl.BlockSpec((_BQ, 4 * D_IN), lambda i: (i, 0)),
        pl.BlockSpec((2, 4 * D_IN, 4 * F), lambda i: (0, 0, 0)),
        pl.BlockSpec((2, 4 * F), lambda i: (0, 0)),
    ],
    out_specs=[
        pl.BlockSpec((_BQ, 4 * F), lambda i: (i, 0)),
        pl.BlockSpec((_BQ, 4 * F), lambda i: (i, 0)),
    ],
    out_shape=[
        jax.ShapeDtypeStruct((NQ, 4 * F), jnp.float32),
        jax.ShapeDtypeStruct((NQ, 4 * F), jnp.float32),
    ],
)


def _dense_body(al_ref, ah_ref, hl_ref, hh_ref, w_ref, ol_ref, oh_ref):
    ml = (1.0 - ALPHA) * al_ref[...] + ALPHA * hl_ref[...]
    mh = (1.0 - ALPHA) * ah_ref[...] + ALPHA * hh_ref[...]
    rl = jnp.dot(ml, w_ref[0], preferred_element_type=jnp.float32)
    rl = rl + jnp.dot(mh, w_ref[1], preferred_element_type=jnp.float32)
    rh = jnp.dot(ml, w_ref[2], preferred_element_type=jnp.float32)
    rh = rh + jnp.dot(mh, w_ref[3], preferred_element_type=jnp.float32)
    ol_ref[...] = jnp.maximum(rl, 0.0)
    oh_ref[...] = jnp.maximum(rh, 0.0)


_dense = pl.pallas_call(
    _dense_body,
    grid=(_NBQ,),
    in_specs=[
        pl.BlockSpec((_BQ, 4 * F), lambda i: (i, 0)),
        pl.BlockSpec((_BQ, 4 * F), lambda i: (i, 0)),
        pl.BlockSpec((_BQ, 4 * F), lambda i: (i, 0)),
        pl.BlockSpec((_BQ, 4 * F), lambda i: (i, 0)),
        pl.BlockSpec((4, 4 * F, 4 * F), lambda i: (0, 0, 0)),
    ],
    out_specs=[
        pl.BlockSpec((_BQ, 4 * F), lambda i: (i, 0)),
        pl.BlockSpec((_BQ, 4 * F), lambda i: (i, 0)),
    ],
    out_shape=[
        jax.ShapeDtypeStruct((NQ, 4 * F), jnp.float32),
        jax.ShapeDtypeStruct((NQ, 4 * F), jnp.float32),
    ],
)


def _outp_body(hl_ref, hh_ref, w_ref, b_ref, o_ref):
    r = jnp.dot(hl_ref[...], w_ref[0], preferred_element_type=jnp.float32)
    r = r + jnp.dot(hh_ref[...], w_ref[1], preferred_element_type=jnp.float32)
    o_ref[...] = r + b_ref[...]


_outp = pl.pallas_call(
    _outp_body,
    grid=(_NBQ,),
    in_specs=[
        pl.BlockSpec((_BQ, 4 * F), lambda i: (i, 0)),
        pl.BlockSpec((_BQ, 4 * F), lambda i: (i, 0)),
        pl.BlockSpec((2, 4 * F, 4 * HID), lambda i: (0, 0, 0)),
        pl.BlockSpec((1, 4 * HID), lambda i: (0, 0)),
    ],
    out_specs=pl.BlockSpec((_BQ, 4 * HID), lambda i: (i, 0)),
    out_shape=jax.ShapeDtypeStruct((NQ, 4 * HID), jnp.float32),
)


def kernel(x, edge_index, W0, b0, Wconvs, Wout, bout):
    ei = edge_index.astype(jnp.int32)
    srcr = ei[0].reshape(N_EDGES // DB, DB)
    dstr = ei[1].reshape(N_EDGES // DB, DB)

    betas = jnp.asarray(
        [math.log(THETA / (l + 1) + 1.0) for l in range(N_LAYERS)], jnp.float32
    )
    eye = jnp.eye(HID, dtype=jnp.float32)
    wp = (1.0 - betas)[:, None, None] * eye + betas[:, None, None] * Wconvs
    eye4 = jnp.eye(4, dtype=jnp.float32)
    wpb = jnp.stack(
        [wp[:, :F, :F], wp[:, F:, :F], wp[:, :F, F:], wp[:, F:, F:]], axis=1
    )
    wd = jnp.einsum("ab,lkij->lkaibj", eye4, wpb).reshape(
        N_LAYERS, 4, 4 * F, 4 * F
    )
    wpj = jnp.stack(
        [jnp.kron(eye4, W0[:, :F]), jnp.kron(eye4, W0[:, F:])]
    )
    bpj = jnp.stack([jnp.tile(b0[:F], 4), jnp.tile(b0[F:], 4)])
    wo4 = jnp.stack([jnp.kron(eye4, Wout[:F]), jnp.kron(eye4, Wout[F:])])
    bo4 = jnp.tile(bout, 4).reshape(1, 4 * HID)

    x4 = x.reshape(NQ, 4 * D_IN)
    hl4, hh4 = _proj(x4, wpj, bpj)
    h0l4, h0h4 = hl4, hh4
    for l in range(N_LAYERS):
        al, ah = _segsum(
            hl4.reshape(N_NODES, F), hh4.reshape(N_NODES, F), srcr, dstr
        )
        hl4, hh4 = _dense(
            al.reshape(NQ, 4 * F), ah.reshape(NQ, 4 * F), h0l4, h0h4, wd[l]
        )
    return _outp(hl4, hh4, wo4, bo4).reshape(N_NODES, HID)


# speedup vs baseline: 18.9131x; 1.0172x over previous
import functools
import math

import jax
import jax.numpy as jnp
from jax import lax
from jax.experimental import pallas as pl
from jax.experimental.pallas import tpu as pltpu
from jax.experimental.pallas import tpu_sc as plsc

N_NODES = 50000
N_EDGES = 800000
D_IN = 128
HID = 64
N_LAYERS = 8
ALPHA = 0.1
THETA = 0.5

F = HID // 2
TILES = 16
EPT = N_EDGES // TILES
DB = 125
DPC = 2
CH = DB * DPC
CPB = 40
IBR = 4 * DPC
NIB = CPB // 4
NBODY = EPT // (CH * CPB)
RPT_A = 3128
RPT_LAST = N_NODES - 15 * RPT_A
DST_OFF = N_EDGES // DB

_mesh = plsc.VectorSubcoreMesh(core_axis_name="c", subcore_axis_name="s")


def _segsum_body(hlo, hhi, sd, s0lo, s0hi, outlo, outhi,
                 idx_s0, idx_s1, idx_d0, idx_d1, rows0, rows1, rows2, acc,
                 gsem0, gsem1, gsem2, ssem0, ssem1, ssem2, isem0, isem1):
    c = lax.axis_index("c")
    s = lax.axis_index("s")
    base = pl.multiple_of(s * RPT_A, 8)
    isrc = (idx_s0, idx_s1)
    idst = (idx_d0, idx_d1)
    rbuf = (rows0, rows1, rows2)
    gsem = (gsem0, gsem1, gsem2)
    ssem = (ssem0, ssem1, ssem2)
    isem = (isem0, isem1)

    @pl.when(jnp.logical_and(c == 0, s < TILES - 1))
    def _():
        pltpu.sync_copy(s0lo.at[pl.ds(base, RPT_A)], acc.at[pl.ds(base, RPT_A)])

    @pl.when(jnp.logical_and(c == 0, s == TILES - 1))
    def _():
        pltpu.sync_copy(s0lo.at[pl.ds(base, RPT_LAST)], acc.at[pl.ds(base, RPT_LAST)])

    @pl.when(jnp.logical_and(c == 1, s < TILES - 1))
    def _():
        pltpu.sync_copy(s0hi.at[pl.ds(base, RPT_A)], acc.at[pl.ds(base, RPT_A)])

    @pl.when(jnp.logical_and(c == 1, s == TILES - 1))
    def _():
        pltpu.sync_copy(s0hi.at[pl.ds(base, RPT_LAST)], acc.at[pl.ds(base, RPT_LAST)])

    plsc.subcore_barrier()

    def _body(t, _):
        def idx_fire(m):
            ib = m & 1
            r0 = pl.multiple_of((s * (NIB * NBODY) + t * NIB + m) * IBR, 8)
            return [
                pltpu.async_copy(sd.at[pl.ds(r0, IBR)], isrc[ib], isem[ib]),
                pltpu.async_copy(sd.at[pl.ds(DST_OFF + r0, IBR)], idst[ib], isem[ib]),
            ]

        ih = {0: idx_fire(0), 1: idx_fire(1)}

        def g_fire(cc, rb):
            m, k = divmod(cc, 4)
            ib = m & 1
            out = []

            @pl.when(c == 0)
            def _():
                for j in range(DPC):
                    out.append(pltpu.async_copy(
                        hlo.at[isrc[ib].at[k * DPC + j]],
                        rbuf[rb].at[pl.ds(j * DB, DB)], gsem[rb]))

            @pl.when(c == 1)
            def _():
                for j in range(DPC):
                    out.append(pltpu.async_copy(
                        hhi.at[isrc[ib].at[k * DPC + j]],
                        rbuf[rb].at[pl.ds(j * DB, DB)], gsem[rb]))

            return out

        def g_wait(rb):
            for j in range(DPC):
                pltpu.make_async_copy(
                    hlo.at[isrc[0].at[0]],
                    rbuf[rb].at[pl.ds(j * DB, DB)], gsem[rb]).wait()

        def s_fire(cc, rb):
            m, k = divmod(cc, 4)
            ib = m & 1
            return [pltpu.async_copy(
                rbuf[rb].at[pl.ds(j * DB, DB)],
                acc.at[idst[ib].at[k * DPC + j]], ssem[rb], add=True)
                for j in range(DPC)]

        def s_wait(rb):
            for j in range(DPC):
                pltpu.make_async_copy(
                    rbuf[rb].at[pl.ds(j * DB, DB)],
                    acc.at[idst[0].at[0]], ssem[rb]).wait()

        for cp in ih[0]:
            cp.wait()
        g_fire(0, 0)
        g_fire(1, 1)
        for cc in range(CPB):
            rb = cc % 3
            if cc >= 1:
                s_wait((cc - 1) % 3)
            if cc + 2 < CPB:
                if (cc + 2) % 4 == 0:
                    for cp in ih[(cc + 2) // 4]:
                        cp.wait()
                if cc % 4 == 0 and 0 < cc and cc // 4 + 1 < NIB:
                    ih[cc // 4 + 1] = idx_fire(cc // 4 + 1)
                g_fire(cc + 2, (cc + 2) % 3)
            g_wait(rb)
            s_fire(cc, rb)
        s_wait((CPB - 1) % 3)
        return _

    lax.fori_loop(0, NBODY, _body, None)
    plsc.subcore_barrier()

    @pl.when(jnp.logical_and(c == 0, s < TILES - 1))
    def _():
        pltpu.sync_copy(acc.at[pl.ds(base, RPT_A)], outlo.at[pl.ds(base, RPT_A)])

    @pl.when(jnp.logical_and(c == 0, s == TILES - 1))
    def _():
        pltpu.sync_copy(acc.at[pl.ds(base, RPT_LAST)], outlo.at[pl.ds(base, RPT_LAST)])

    @pl.when(jnp.logical_and(c == 1, s < TILES - 1))
    def _():
        pltpu.sync_copy(acc.at[pl.ds(base, RPT_A)], outhi.at[pl.ds(base, RPT_A)])

    @pl.when(jnp.logical_and(c == 1, s == TILES - 1))
    def _():
        pltpu.sync_copy(acc.at[pl.ds(base, RPT_LAST)], outhi.at[pl.ds(base, RPT_LAST)])


_segsum = pl.kernel(
    _segsum_body,
    out_type=[
        jax.ShapeDtypeStruct((N_NODES, F), jnp.float32),
        jax.ShapeDtypeStruct((N_NODES, F), jnp.float32),
    ],
    mesh=_mesh,
    compiler_params=pltpu.CompilerParams(use_tc_tiling_on_sc=False),
    scratch_types=[
        pltpu.VMEM((IBR, DB), jnp.int32),
        pltpu.VMEM((IBR, DB), jnp.int32),
        pltpu.VMEM((IBR, DB), jnp.int32),
        pltpu.VMEM((IBR, DB), jnp.int32),
        pltpu.VMEM((CH, F), jnp.float32),
        pltpu.VMEM((CH, F), jnp.float32),
        pltpu.VMEM((CH, F), jnp.float32),
        pltpu.VMEM_SHARED((N_NODES, F), jnp.float32),
        pltpu.SemaphoreType.DMA,
        pltpu.SemaphoreType.DMA,
        pltpu.SemaphoreType.DMA,
        pltpu.SemaphoreType.DMA,
        pltpu.SemaphoreType.DMA,
        pltpu.SemaphoreType.DMA,
        pltpu.SemaphoreType.DMA,
        pltpu.SemaphoreType.DMA,
    ],
)


NQ = N_NODES // 4
_BQ = 2000
_NBQ = -(-NQ // _BQ)
_BR = 4 * _BQ
_NB = _NBQ


_S0 = ALPHA / (1.0 - ALPHA)


def _proj_body(x_ref, w_ref, b_ref, ol_ref, oh_ref, sl_ref, sh_ref):
    xb = x_ref[...]
    rl = jnp.dot(xb, w_ref[0], preferred_element_type=jnp.float32)
    rh = jnp.dot(xb, w_ref[1], preferred_element_type=jnp.float32)
    hl = jnp.maximum(rl + b_ref[0:1, :], 0.0)
    hh = jnp.maximum(rh + b_ref[1:2, :], 0.0)
    ol_ref[...] = hl
    oh_ref[...] = hh
    sl_ref[...] = _S0 * hl
    sh_ref[...] = _S0 * hh


_proj = pl.pallas_call(
    _proj_body,
    grid=(_NBQ,),
    in_specs=[
        pl.BlockSpec((_BQ, 4 * D_IN), lambda i: (i, 0)),
        pl.BlockSpec((2, 4 * D_IN, 4 * F), lambda i: (0, 0, 0)),
        pl.BlockSpec((2, 4 * F), lambda i: (0, 0)),
    ],
    out_specs=[
        pl.BlockSpec((_BQ, 4 * F), lambda i: (i, 0)),
        pl.BlockSpec((_BQ, 4 * F), lambda i: (i, 0)),
        pl.BlockSpec((_BQ, 4 * F), lambda i: (i, 0)),
        pl.BlockSpec((_BQ, 4 * F), lambda i: (i, 0)),
    ],
    out_shape=[
        jax.ShapeDtypeStruct((NQ, 4 * F), jnp.float32),
        jax.ShapeDtypeStruct((NQ, 4 * F), jnp.float32),
        jax.ShapeDtypeStruct((NQ, 4 * F), jnp.float32),
        jax.ShapeDtypeStruct((NQ, 4 * F), jnp.float32),
    ],
)


def _dense_body(al_ref, ah_ref, w_ref, ol_ref, oh_ref):
    ml = al_ref[...]
    mh = ah_ref[...]
    rl = jnp.dot(ml, w_ref[0], preferred_element_type=jnp.float32)
    rl = rl + jnp.dot(mh, w_ref[1], preferred_element_type=jnp.float32)
    rh = jnp.dot(ml, w_ref[2], preferred_element_type=jnp.float32)
    rh = rh + jnp.dot(mh, w_ref[3], preferred_element_type=jnp.float32)
    ol_ref[...] = jnp.maximum(rl, 0.0)
    oh_ref[...] = jnp.maximum(rh, 0.0)


_dense = pl.pallas_call(
    _dense_body,
    grid=(_NBQ,),
    in_specs=[
        pl.BlockSpec((_BQ, 4 * F), lambda i: (i, 0)),
        pl.BlockSpec((_BQ, 4 * F), lambda i: (i, 0)),
        pl.BlockSpec((4, 4 * F, 4 * F), lambda i: (0, 0, 0)),
    ],
    out_specs=[
        pl.BlockSpec((_BQ, 4 * F), lambda i: (i, 0)),
        pl.BlockSpec((_BQ, 4 * F), lambda i: (i, 0)),
    ],
    out_shape=[
        jax.ShapeDtypeStruct((NQ, 4 * F), jnp.float32),
        jax.ShapeDtypeStruct((NQ, 4 * F), jnp.float32),
    ],
)


def _outp_body(hl_ref, hh_ref, w_ref, b_ref, o_ref):
    r = jnp.dot(hl_ref[...], w_ref[0], preferred_element_type=jnp.float32)
    r = r + jnp.dot(hh_ref[...], w_ref[1], preferred_element_type=jnp.float32)
    o_ref[...] = r + b_ref[...]


_outp = pl.pallas_call(
    _outp_body,
    grid=(_NBQ,),
    in_specs=[
        pl.BlockSpec((_BQ, 4 * F), lambda i: (i, 0)),
        pl.BlockSpec((_BQ, 4 * F), lambda i: (i, 0)),
        pl.BlockSpec((2, 4 * F, 4 * HID), lambda i: (0, 0, 0)),
        pl.BlockSpec((1, 4 * HID), lambda i: (0, 0)),
    ],
    out_specs=pl.BlockSpec((_BQ, 4 * HID), lambda i: (i, 0)),
    out_shape=jax.ShapeDtypeStruct((NQ, 4 * HID), jnp.float32),
)


def kernel(x, edge_index, W0, b0, Wconvs, Wout, bout):
    sd = edge_index.astype(jnp.int32).reshape(2 * (N_EDGES // DB), DB)

    betas = jnp.asarray(
        [math.log(THETA / (l + 1) + 1.0) for l in range(N_LAYERS)], jnp.float32
    )
    eye = jnp.eye(HID, dtype=jnp.float32)
    wp = (1.0 - betas)[:, None, None] * eye + betas[:, None, None] * Wconvs
    wp = (1.0 - ALPHA) * wp
    eye4 = jnp.eye(4, dtype=jnp.float32)
    wpb = jnp.stack(
        [wp[:, :F, :F], wp[:, F:, :F], wp[:, :F, F:], wp[:, F:, F:]], axis=1
    )
    wd = jnp.einsum("ab,lkij->lkaibj", eye4, wpb).reshape(
        N_LAYERS, 4, 4 * F, 4 * F
    )
    wpj = jnp.stack(
        [jnp.kron(eye4, W0[:, :F]), jnp.kron(eye4, W0[:, F:])]
    )
    bpj = jnp.stack([jnp.tile(b0[:F], 4), jnp.tile(b0[F:], 4)])
    wo4 = jnp.stack([jnp.kron(eye4, Wout[:F]), jnp.kron(eye4, Wout[F:])])
    bo4 = jnp.tile(bout, 4).reshape(1, 4 * HID)

    x4 = x.reshape(NQ, 4 * D_IN)
    hl4, hh4, s0l4, s0h4 = _proj(x4, wpj, bpj)
    s0l = s0l4.reshape(N_NODES, F)
    s0h = s0h4.reshape(N_NODES, F)
    for l in range(N_LAYERS):
        al, ah = _segsum(
            hl4.reshape(N_NODES, F), hh4.reshape(N_NODES, F), sd, s0l, s0h
        )
        hl4, hh4 = _dense(al.reshape(NQ, 4 * F), ah.reshape(NQ, 4 * F), wd[l])
    return _outp(hl4, hh4, wo4, bo4).reshape(N_NODES, HID)


# speedup vs baseline: 19.2756x; 1.0192x over previous
import functools
import math

import jax
import jax.numpy as jnp
from jax import lax
from jax.experimental import pallas as pl
from jax.experimental.pallas import tpu as pltpu
from jax.experimental.pallas import tpu_sc as plsc

N_NODES = 50000
N_EDGES = 800000
D_IN = 128
HID = 64
N_LAYERS = 8
ALPHA = 0.1
THETA = 0.5

F = HID // 2
TILES = 16
EPT = N_EDGES // TILES
DB = 125
DPC = 2
CH = DB * DPC
CPB = 40
IBR = 4 * DPC
NIB = CPB // 4
NBODY = EPT // (CH * CPB)
RPT_A = 3128
RPT_LAST = N_NODES - 15 * RPT_A
DST_OFF = N_EDGES // DB

_mesh = plsc.VectorSubcoreMesh(core_axis_name="c", subcore_axis_name="s")


def _segsum_body(hlo, hhi, sd, s0lo, s0hi, outlo, outhi,
                 idx_s0, idx_s1, idx_d0, idx_d1, rows0, rows1, rows2, acc,
                 gsem0, gsem1, gsem2, ssem0, ssem1, ssem2, isem0, isem1):
    c = lax.axis_index("c")
    s = lax.axis_index("s")
    base = pl.multiple_of(s * RPT_A, 8)
    isrc = (idx_s0, idx_s1)
    idst = (idx_d0, idx_d1)
    rbuf = (rows0, rows1, rows2)
    gsem = (gsem0, gsem1, gsem2)
    ssem = (ssem0, ssem1, ssem2)
    isem = (isem0, isem1)

    def idx_fire(m, tt):
        ib = m & 1
        r0 = pl.multiple_of((s * (NIB * NBODY) + tt * NIB + m) * IBR, 8)
        return [
            pltpu.async_copy(sd.at[pl.ds(r0, IBR)], isrc[ib], isem[ib]),
            pltpu.async_copy(sd.at[pl.ds(DST_OFF + r0, IBR)], idst[ib], isem[ib]),
        ]

    def g_fire(cc, rb):
        m, k = divmod(cc, 4)
        ib = m & 1

        @pl.when(c == 0)
        def _():
            for j in range(DPC):
                pltpu.async_copy(
                    hlo.at[isrc[ib].at[k * DPC + j]],
                    rbuf[rb].at[pl.ds(j * DB, DB)], gsem[rb])

        @pl.when(c == 1)
        def _():
            for j in range(DPC):
                pltpu.async_copy(
                    hhi.at[isrc[ib].at[k * DPC + j]],
                    rbuf[rb].at[pl.ds(j * DB, DB)], gsem[rb])

    def g_wait(rb):
        for j in range(DPC):
            pltpu.make_async_copy(
                hlo.at[isrc[0].at[0]],
                rbuf[rb].at[pl.ds(j * DB, DB)], gsem[rb]).wait()

    def s_fire(cc, rb):
        m, k = divmod(cc, 4)
        ib = m & 1
        for j in range(DPC):
            pltpu.async_copy(
                rbuf[rb].at[pl.ds(j * DB, DB)],
                acc.at[idst[ib].at[k * DPC + j]], ssem[rb], add=True)

    def s_wait(rb):
        for j in range(DPC):
            pltpu.make_async_copy(
                rbuf[rb].at[pl.ds(j * DB, DB)],
                acc.at[idst[0].at[0]], ssem[rb]).wait()

    def prefetch(tt):
        for cp in idx_fire(0, tt) + idx_fire(1, tt):
            cp.wait()
        g_fire(0, 0)
        g_fire(1, 1)

    prefetch(0)

    @pl.when(jnp.logical_and(c == 0, s < TILES - 1))
    def _():
        pltpu.sync_copy(s0lo.at[pl.ds(base, RPT_A)], acc.at[pl.ds(base, RPT_A)])

    @pl.when(jnp.logical_and(c == 0, s == TILES - 1))
    def _():
        pltpu.sync_copy(s0lo.at[pl.ds(base, RPT_LAST)], acc.at[pl.ds(base, RPT_LAST)])

    @pl.when(jnp.logical_and(c == 1, s < TILES - 1))
    def _():
        pltpu.sync_copy(s0hi.at[pl.ds(base, RPT_A)], acc.at[pl.ds(base, RPT_A)])

    @pl.when(jnp.logical_and(c == 1, s == TILES - 1))
    def _():
        pltpu.sync_copy(s0hi.at[pl.ds(base, RPT_LAST)], acc.at[pl.ds(base, RPT_LAST)])

    plsc.subcore_barrier()

    def _body(t, _):
        ih = {}
        for cc in range(CPB):
            rb = cc % 3
            if cc >= 1:
                s_wait((cc - 1) % 3)
            if cc + 2 < CPB:
                if (cc + 2) % 4 == 0 and (cc + 2) // 4 >= 2:
                    for cp in ih[(cc + 2) // 4]:
                        cp.wait()
                if cc % 4 == 0 and 0 < cc and cc // 4 + 1 < NIB:
                    ih[cc // 4 + 1] = idx_fire(cc // 4 + 1, t)
                g_fire(cc + 2, (cc + 2) % 3)
            g_wait(rb)
            s_fire(cc, rb)
        s_wait((CPB - 1) % 3)

        @pl.when(t + 1 < NBODY)
        def _():
            prefetch(t + 1)

        return _

    lax.fori_loop(0, NBODY, _body, None)
    plsc.subcore_barrier()

    @pl.when(jnp.logical_and(c == 0, s < TILES - 1))
    def _():
        pltpu.sync_copy(acc.at[pl.ds(base, RPT_A)], outlo.at[pl.ds(base, RPT_A)])

    @pl.when(jnp.logical_and(c == 0, s == TILES - 1))
    def _():
        pltpu.sync_copy(acc.at[pl.ds(base, RPT_LAST)], outlo.at[pl.ds(base, RPT_LAST)])

    @pl.when(jnp.logical_and(c == 1, s < TILES - 1))
    def _():
        pltpu.sync_copy(acc.at[pl.ds(base, RPT_A)], outhi.at[pl.ds(base, RPT_A)])

    @pl.when(jnp.logical_and(c == 1, s == TILES - 1))
    def _():
        pltpu.sync_copy(acc.at[pl.ds(base, RPT_LAST)], outhi.at[pl.ds(base, RPT_LAST)])


_segsum = pl.kernel(
    _segsum_body,
    out_type=[
        jax.ShapeDtypeStruct((N_NODES, F), jnp.float32),
        jax.ShapeDtypeStruct((N_NODES, F), jnp.float32),
    ],
    mesh=_mesh,
    compiler_params=pltpu.CompilerParams(use_tc_tiling_on_sc=False),
    scratch_types=[
        pltpu.VMEM((IBR, DB), jnp.int32),
        pltpu.VMEM((IBR, DB), jnp.int32),
        pltpu.VMEM((IBR, DB), jnp.int32),
        pltpu.VMEM((IBR, DB), jnp.int32),
        pltpu.VMEM((CH, F), jnp.float32),
        pltpu.VMEM((CH, F), jnp.float32),
        pltpu.VMEM((CH, F), jnp.float32),
        pltpu.VMEM_SHARED((N_NODES, F), jnp.float32),
        pltpu.SemaphoreType.DMA,
        pltpu.SemaphoreType.DMA,
        pltpu.SemaphoreType.DMA,
        pltpu.SemaphoreType.DMA,
        pltpu.SemaphoreType.DMA,
        pltpu.SemaphoreType.DMA,
        pltpu.SemaphoreType.DMA,
        pltpu.SemaphoreType.DMA,
    ],
)


NQ = N_NODES // 4
_BQ = 2000
_NBQ = -(-NQ // _BQ)
_BR = 4 * _BQ
_NB = _NBQ


_S0 = ALPHA / (1.0 - ALPHA)


def _proj_body(x_ref, w_ref, b_ref, ol_ref, oh_ref, sl_ref, sh_ref):
    xb = x_ref[...]
    rl = jnp.dot(xb, w_ref[0], preferred_element_type=jnp.float32)
    rh = jnp.dot(xb, w_ref[1], preferred_element_type=jnp.float32)
    hl = jnp.maximum(rl + b_ref[0:1, :], 0.0)
    hh = jnp.maximum(rh + b_ref[1:2, :], 0.0)
    ol_ref[...] = hl
    oh_ref[...] = hh
    sl_ref[...] = _S0 * hl
    sh_ref[...] = _S0 * hh


_proj = pl.pallas_call(
    _proj_body,
    grid=(_NBQ,),
    in_specs=[
        pl.BlockSpec((_BQ, 4 * D_IN), lambda i: (i, 0)),
        pl.BlockSpec((2, 4 * D_IN, 4 * F), lambda i: (0, 0, 0)),
        pl.BlockSpec((2, 4 * F), lambda i: (0, 0)),
    ],
    out_specs=[
        pl.BlockSpec((_BQ, 4 * F), lambda i: (i, 0)),
        pl.BlockSpec((_BQ, 4 * F), lambda i: (i, 0)),
        pl.BlockSpec((_BQ, 4 * F), lambda i: (i, 0)),
        pl.BlockSpec((_BQ, 4 * F), lambda i: (i, 0)),
    ],
    out_shape=[
        jax.ShapeDtypeStruct((NQ, 4 * F), jnp.float32),
        jax.ShapeDtypeStruct((NQ, 4 * F), jnp.float32),
        jax.ShapeDtypeStruct((NQ, 4 * F), jnp.float32),
        jax.ShapeDtypeStruct((NQ, 4 * F), jnp.float32),
    ],
)


def _dense_body(al_ref, ah_ref, w_ref, ol_ref, oh_ref):
    ml = al_ref[...]
    mh = ah_ref[...]
    rl = jnp.dot(ml, w_ref[0], preferred_element_type=jnp.float32)
    rl = rl + jnp.dot(mh, w_ref[1], preferred_element_type=jnp.float32)
    rh = jnp.dot(ml, w_ref[2], preferred_element_type=jnp.float32)
    rh = rh + jnp.dot(mh, w_ref[3], preferred_element_type=jnp.float32)
    ol_ref[...] = jnp.maximum(rl, 0.0)
    oh_ref[...] = jnp.maximum(rh, 0.0)


_dense = pl.pallas_call(
    _dense_body,
    grid=(_NBQ,),
    in_specs=[
        pl.BlockSpec((_BQ, 4 * F), lambda i: (i, 0)),
        pl.BlockSpec((_BQ, 4 * F), lambda i: (i, 0)),
        pl.BlockSpec((4, 4 * F, 4 * F), lambda i: (0, 0, 0)),
    ],
    out_specs=[
        pl.BlockSpec((_BQ, 4 * F), lambda i: (i, 0)),
        pl.BlockSpec((_BQ, 4 * F), lambda i: (i, 0)),
    ],
    out_shape=[
        jax.ShapeDtypeStruct((NQ, 4 * F), jnp.float32),
        jax.ShapeDtypeStruct((NQ, 4 * F), jnp.float32),
    ],
)


def _outp_body(hl_ref, hh_ref, w_ref, b_ref, o_ref):
    r = jnp.dot(hl_ref[...], w_ref[0], preferred_element_type=jnp.float32)
    r = r + jnp.dot(hh_ref[...], w_ref[1], preferred_element_type=jnp.float32)
    o_ref[...] = r + b_ref[...]


_outp = pl.pallas_call(
    _outp_body,
    grid=(_NBQ,),
    in_specs=[
        pl.BlockSpec((_BQ, 4 * F), lambda i: (i, 0)),
        pl.BlockSpec((_BQ, 4 * F), lambda i: (i, 0)),
        pl.BlockSpec((2, 4 * F, 4 * HID), lambda i: (0, 0, 0)),
        pl.BlockSpec((1, 4 * HID), lambda i: (0, 0)),
    ],
    out_specs=pl.BlockSpec((_BQ, 4 * HID), lambda i: (i, 0)),
    out_shape=jax.ShapeDtypeStruct((NQ, 4 * HID), jnp.float32),
)


def kernel(x, edge_index, W0, b0, Wconvs, Wout, bout):
    sd = edge_index.astype(jnp.int32).reshape(2 * (N_EDGES // DB), DB)

    betas = jnp.asarray(
        [math.log(THETA / (l + 1) + 1.0) for l in range(N_LAYERS)], jnp.float32
    )
    eye = jnp.eye(HID, dtype=jnp.float32)
    wp = (1.0 - betas)[:, None, None] * eye + betas[:, None, None] * Wconvs
    wp = (1.0 - ALPHA) * wp
    eye4 = jnp.eye(4, dtype=jnp.float32)
    wpb = jnp.stack(
        [wp[:, :F, :F], wp[:, F:, :F], wp[:, :F, F:], wp[:, F:, F:]], axis=1
    )
    wd = jnp.einsum("ab,lkij->lkaibj", eye4, wpb).reshape(
        N_LAYERS, 4, 4 * F, 4 * F
    )
    wpj = jnp.stack(
        [jnp.kron(eye4, W0[:, :F]), jnp.kron(eye4, W0[:, F:])]
    )
    bpj = jnp.stack([jnp.tile(b0[:F], 4), jnp.tile(b0[F:], 4)])
    wo4 = jnp.stack([jnp.kron(eye4, Wout[:F]), jnp.kron(eye4, Wout[F:])])
    bo4 = jnp.tile(bout, 4).reshape(1, 4 * HID)

    x4 = x.reshape(NQ, 4 * D_IN)
    hl4, hh4, s0l4, s0h4 = _proj(x4, wpj, bpj)
    s0l = s0l4.reshape(N_NODES, F)
    s0h = s0h4.reshape(N_NODES, F)
    for l in range(N_LAYERS):
        al, ah = _segsum(
            hl4.reshape(N_NODES, F), hh4.reshape(N_NODES, F), sd, s0l, s0h
        )
        hl4, hh4 = _dense(al.reshape(NQ, 4 * F), ah.reshape(NQ, 4 * F), wd[l])
    return _outp(hl4, hh4, wo4, bo4).reshape(N_NODES, HID)


# speedup vs baseline: 19.4011x; 1.0065x over previous
import math

import jax
import jax.numpy as jnp
from jax import lax
from jax.experimental import pallas as pl
from jax.experimental.pallas import tpu as pltpu
from jax.experimental.pallas import tpu_sc as plsc

N_NODES = 50000
N_EDGES = 800000
D_IN = 128
HID = 64
N_LAYERS = 8
ALPHA = 0.1
THETA = 0.5

F = HID // 2
TILES = 16
EPT = N_EDGES // TILES
DB = 125
DPC = 2
CH = DB * DPC
CPB = 40
IBR = 4 * DPC
NIB = CPB // 4
NBODY = EPT // (CH * CPB)
RPT_A = 3128
RPT_LAST = N_NODES - 15 * RPT_A
DST_OFF = N_EDGES // DB

_mesh = plsc.VectorSubcoreMesh(core_axis_name="c", subcore_axis_name="s")


def _segsum_body(hlo, hhi, sd, s0lo, s0hi, outlo, outhi,
                 idx_s0, idx_s1, idx_d0, idx_d1, rows0, rows1, rows2, acc,
                 gsem0, gsem1, gsem2, ssem0, ssem1, ssem2, isem0, isem1):
    c = lax.axis_index("c")
    s = lax.axis_index("s")
    base = pl.multiple_of(s * RPT_A, 8)
    isrc = (idx_s0, idx_s1)
    idst = (idx_d0, idx_d1)
    rbuf = (rows0, rows1, rows2)
    gsem = (gsem0, gsem1, gsem2)
    ssem = (ssem0, ssem1, ssem2)
    isem = (isem0, isem1)

    def idx_fire(m, tt):
        ib = m & 1
        r0 = pl.multiple_of((s * (NIB * NBODY) + tt * NIB + m) * IBR, 8)
        return [
            pltpu.async_copy(sd.at[pl.ds(r0, IBR)], isrc[ib], isem[ib]),
            pltpu.async_copy(sd.at[pl.ds(DST_OFF + r0, IBR)], idst[ib], isem[ib]),
        ]

    def g_fire(cc, rb):
        m, k = divmod(cc, 4)
        ib = m & 1

        @pl.when(c == 0)
        def _():
            for j in range(DPC):
                pltpu.async_copy(
                    hlo.at[isrc[ib].at[k * DPC + j]],
                    rbuf[rb].at[pl.ds(j * DB, DB)], gsem[rb])

        @pl.when(c == 1)
        def _():
            for j in range(DPC):
                pltpu.async_copy(
                    hhi.at[isrc[ib].at[k * DPC + j]],
                    rbuf[rb].at[pl.ds(j * DB, DB)], gsem[rb])

    def g_wait(rb):
        for j in range(DPC):
            pltpu.make_async_copy(
                hlo.at[isrc[0].at[0]],
                rbuf[rb].at[pl.ds(j * DB, DB)], gsem[rb]).wait()

    def s_fire(cc, rb):
        m, k = divmod(cc, 4)
        ib = m & 1
        for j in range(DPC):
            pltpu.async_copy(
                rbuf[rb].at[pl.ds(j * DB, DB)],
                acc.at[idst[ib].at[k * DPC + j]], ssem[rb], add=True)

    def s_wait(rb):
        for j in range(DPC):
            pltpu.make_async_copy(
                rbuf[rb].at[pl.ds(j * DB, DB)],
                acc.at[idst[0].at[0]], ssem[rb]).wait()

    def prefetch(tt):
        for cp in idx_fire(0, tt) + idx_fire(1, tt):
            cp.wait()
        g_fire(0, 0)
        g_fire(1, 1)

    prefetch(0)

    @pl.when(jnp.logical_and(c == 0, s < TILES - 1))
    def _():
        pltpu.sync_copy(s0lo.at[pl.ds(base, RPT_A)], acc.at[pl.ds(base, RPT_A)])

    @pl.when(jnp.logical_and(c == 0, s == TILES - 1))
    def _():
        pltpu.sync_copy(s0lo.at[pl.ds(base, RPT_LAST)], acc.at[pl.ds(base, RPT_LAST)])

    @pl.when(jnp.logical_and(c == 1, s < TILES - 1))
    def _():
        pltpu.sync_copy(s0hi.at[pl.ds(base, RPT_A)], acc.at[pl.ds(base, RPT_A)])

    @pl.when(jnp.logical_and(c == 1, s == TILES - 1))
    def _():
        pltpu.sync_copy(s0hi.at[pl.ds(base, RPT_LAST)], acc.at[pl.ds(base, RPT_LAST)])

    plsc.subcore_barrier()

    def _body(t, _):
        ih = {}
        for cc in range(CPB):
            rb = cc % 3
            if cc >= 1:
                s_wait((cc - 1) % 3)
            if cc + 2 < CPB:
                if (cc + 2) % 4 == 0 and (cc + 2) // 4 >= 2:
                    for cp in ih[(cc + 2) // 4]:
                        cp.wait()
                if cc % 4 == 0 and 0 < cc and cc // 4 + 1 < NIB:
                    ih[cc // 4 + 1] = idx_fire(cc // 4 + 1, t)
                g_fire(cc + 2, (cc + 2) % 3)
            g_wait(rb)
            s_fire(cc, rb)
        s_wait((CPB - 1) % 3)

        @pl.when(t + 1 < NBODY)
        def _():
            prefetch(t + 1)

        return _

    lax.fori_loop(0, NBODY, _body, None)
    plsc.subcore_barrier()

    @pl.when(jnp.logical_and(c == 0, s < TILES - 1))
    def _():
        pltpu.sync_copy(acc.at[pl.ds(base, RPT_A)], outlo.at[pl.ds(base, RPT_A)])

    @pl.when(jnp.logical_and(c == 0, s == TILES - 1))
    def _():
        pltpu.sync_copy(acc.at[pl.ds(base, RPT_LAST)], outlo.at[pl.ds(base, RPT_LAST)])

    @pl.when(jnp.logical_and(c == 1, s < TILES - 1))
    def _():
        pltpu.sync_copy(acc.at[pl.ds(base, RPT_A)], outhi.at[pl.ds(base, RPT_A)])

    @pl.when(jnp.logical_and(c == 1, s == TILES - 1))
    def _():
        pltpu.sync_copy(acc.at[pl.ds(base, RPT_LAST)], outhi.at[pl.ds(base, RPT_LAST)])


_segsum = pl.kernel(
    _segsum_body,
    out_type=[
        jax.ShapeDtypeStruct((N_NODES, F), jnp.float32),
        jax.ShapeDtypeStruct((N_NODES, F), jnp.float32),
    ],
    mesh=_mesh,
    compiler_params=pltpu.CompilerParams(use_tc_tiling_on_sc=False),
    scratch_types=[
        pltpu.VMEM((IBR, DB), jnp.int32),
        pltpu.VMEM((IBR, DB), jnp.int32),
        pltpu.VMEM((IBR, DB), jnp.int32),
        pltpu.VMEM((IBR, DB), jnp.int32),
        pltpu.VMEM((CH, F), jnp.float32),
        pltpu.VMEM((CH, F), jnp.float32),
        pltpu.VMEM((CH, F), jnp.float32),
        pltpu.VMEM_SHARED((N_NODES, F), jnp.float32),
        pltpu.SemaphoreType.DMA,
        pltpu.SemaphoreType.DMA,
        pltpu.SemaphoreType.DMA,
        pltpu.SemaphoreType.DMA,
        pltpu.SemaphoreType.DMA,
        pltpu.SemaphoreType.DMA,
        pltpu.SemaphoreType.DMA,
        pltpu.SemaphoreType.DMA,
    ],
)


NQ = N_NODES // 4
_BQ = 4000
_NBQ = -(-NQ // _BQ)
_BR = 4 * _BQ
_NB = _NBQ


_S0 = ALPHA / (1.0 - ALPHA)


def _proj_body(x_ref, w_ref, b_ref, ol_ref, oh_ref, sl_ref, sh_ref):
    xb = x_ref[...]
    rl = jnp.dot(xb, w_ref[0], preferred_element_type=jnp.float32)
    rh = jnp.dot(xb, w_ref[1], preferred_element_type=jnp.float32)
    hl = jnp.maximum(rl + b_ref[0:1, :], 0.0)
    hh = jnp.maximum(rh + b_ref[1:2, :], 0.0)
    ol_ref[...] = hl
    oh_ref[...] = hh
    sl_ref[...] = _S0 * hl
    sh_ref[...] = _S0 * hh


_proj = pl.pallas_call(
    _proj_body,
    grid=(_NBQ,),
    in_specs=[
        pl.BlockSpec((_BQ, 4 * D_IN), lambda i: (i, 0)),
        pl.BlockSpec((2, 4 * D_IN, 4 * F), lambda i: (0, 0, 0)),
        pl.BlockSpec((2, 4 * F), lambda i: (0, 0)),
    ],
    out_specs=[
        pl.BlockSpec((_BQ, 4 * F), lambda i: (i, 0)),
        pl.BlockSpec((_BQ, 4 * F), lambda i: (i, 0)),
        pl.BlockSpec((_BQ, 4 * F), lambda i: (i, 0)),
        pl.BlockSpec((_BQ, 4 * F), lambda i: (i, 0)),
    ],
    out_shape=[
        jax.ShapeDtypeStruct((NQ, 4 * F), jnp.float32),
        jax.ShapeDtypeStruct((NQ, 4 * F), jnp.float32),
        jax.ShapeDtypeStruct((NQ, 4 * F), jnp.float32),
        jax.ShapeDtypeStruct((NQ, 4 * F), jnp.float32),
    ],
)


def _dense_body(al_ref, ah_ref, w_ref, ol_ref, oh_ref):
    ml = al_ref[...]
    mh = ah_ref[...]
    rl = jnp.dot(ml, w_ref[0], preferred_element_type=jnp.float32)
    rl = rl + jnp.dot(mh, w_ref[1], preferred_element_type=jnp.float32)
    rh = jnp.dot(ml, w_ref[2], preferred_element_type=jnp.float32)
    rh = rh + jnp.dot(mh, w_ref[3], preferred_element_type=jnp.float32)
    ol_ref[...] = jnp.maximum(rl, 0.0)
    oh_ref[...] = jnp.maximum(rh, 0.0)


_dense = pl.pallas_call(
    _dense_body,
    grid=(_NBQ,),
    in_specs=[
        pl.BlockSpec((_BQ, 4 * F), lambda i: (i, 0)),
        pl.BlockSpec((_BQ, 4 * F), lambda i: (i, 0)),
        pl.BlockSpec((4, 4 * F, 4 * F), lambda i: (0, 0, 0)),
    ],
    out_specs=[
        pl.BlockSpec((_BQ, 4 * F), lambda i: (i, 0)),
        pl.BlockSpec((_BQ, 4 * F), lambda i: (i, 0)),
    ],
    out_shape=[
        jax.ShapeDtypeStruct((NQ, 4 * F), jnp.float32),
        jax.ShapeDtypeStruct((NQ, 4 * F), jnp.float32),
    ],
)


def _outp_body(hl_ref, hh_ref, w_ref, b_ref, o_ref):
    r = jnp.dot(hl_ref[...], w_ref[0], preferred_element_type=jnp.float32)
    r = r + jnp.dot(hh_ref[...], w_ref[1], preferred_element_type=jnp.float32)
    o_ref[...] = r + b_ref[...]


_outp = pl.pallas_call(
    _outp_body,
    grid=(_NBQ,),
    in_specs=[
        pl.BlockSpec((_BQ, 4 * F), lambda i: (i, 0)),
        pl.BlockSpec((_BQ, 4 * F), lambda i: (i, 0)),
        pl.BlockSpec((2, 4 * F, 4 * HID), lambda i: (0, 0, 0)),
        pl.BlockSpec((1, 4 * HID), lambda i: (0, 0)),
    ],
    out_specs=pl.BlockSpec((_BQ, 4 * HID), lambda i: (i, 0)),
    out_shape=jax.ShapeDtypeStruct((NQ, 4 * HID), jnp.float32),
)


def kernel(x, edge_index, W0, b0, Wconvs, Wout, bout):
    sd = edge_index.astype(jnp.int32).reshape(2 * (N_EDGES // DB), DB)

    betas = jnp.asarray(
        [math.log(THETA / (l + 1) + 1.0) for l in range(N_LAYERS)], jnp.float32
    )
    eye = jnp.eye(HID, dtype=jnp.float32)
    wp = (1.0 - betas)[:, None, None] * eye + betas[:, None, None] * Wconvs
    wp = (1.0 - ALPHA) * wp
    eye4 = jnp.eye(4, dtype=jnp.float32)
    wpb = jnp.stack(
        [wp[:, :F, :F], wp[:, F:, :F], wp[:, :F, F:], wp[:, F:, F:]], axis=1
    )
    wd = jnp.einsum("ab,lkij->lkaibj", eye4, wpb).reshape(
        N_LAYERS, 4, 4 * F, 4 * F
    )
    wpj = jnp.stack(
        [jnp.kron(eye4, W0[:, :F]), jnp.kron(eye4, W0[:, F:])]
    )
    bpj = jnp.stack([jnp.tile(b0[:F], 4), jnp.tile(b0[F:], 4)])
    wo4 = jnp.stack([jnp.kron(eye4, Wout[:F]), jnp.kron(eye4, Wout[F:])])
    bo4 = jnp.tile(bout, 4).reshape(1, 4 * HID)

    x4 = x.reshape(NQ, 4 * D_IN)
    hl4, hh4, s0l4, s0h4 = _proj(x4, wpj, bpj)
    s0l = s0l4.reshape(N_NODES, F)
    s0h = s0h4.reshape(N_NODES, F)
    for l in range(N_LAYERS):
        al, ah = _segsum(
            hl4.reshape(N_NODES, F), hh4.reshape(N_NODES, F), sd, s0l, s0h
        )
        hl4, hh4 = _dense(al.reshape(NQ, 4 * F), ah.reshape(NQ, 4 * F), wd[l])
    return _outp(hl4, hh4, wo4, bo4).reshape(N_NODES, HID)
